# baseline (speedup 1.0000x reference)
import jax
import jax.numpy as jnp
from jax.experimental import pallas as pl
from jax.experimental.pallas import tpu as pltpu

_D = 1024
_N = 128
_C0, _C1, _C2 = 20000, 50000, 100000
_TW = 4096
_NT = 25
_RING = 13
_NEG = -1e30
_TH = 4
_TT1 = 12
_EH = _C0 - _TH * _TW
_ET1 = _C1 - _TT1 * _TW
_ET2 = _C2 - (_NT - 1) * _TW


def _is_write(i):
    return ((i >= 5) & (i < 9)) | ((i >= 17) & (i < 25)) | (i >= 37)


def _ctile(i):
    t = jnp.where(i < 5, i,
        jnp.where(i < 9, _TH,
        jnp.where(i < 17, i - 4,
        jnp.where(i < 25, _TT1,
        jnp.where(i < 37, i - 12, _NT - 1)))))
    return jnp.clip(t, 0, _NT - 1)


def _wtile(i):
    t = jnp.where(i < 9, i - 5,
        jnp.where(i < 25, i - 13, i - 25))
    return jnp.clip(t, 0, _NT - 1)


def _w_map(i):
    return (_ctile(i), 0)


def _b_map(i):
    return (0, _ctile(i))


def _o_map(i):
    o = jnp.where(i < 5, 0,
        jnp.where(i < 9, i - 5,
        jnp.where(i < 17, _TH,
        jnp.where(i < 25, i - 13,
        jnp.where(i < 37, _TT1, i - 25)))))
    return (o, 0)


def _body(x_ref, w_ref, b_ref, cw_ref, cb_ref, o_ref,
          lscr, mscr, sscr, zh_scr, z1_scr, z2_scr, cl_scr):
    i = pl.program_id(0)
    wr = _is_write(i)

    def _seg_init(vals):
        tmax = jnp.max(vals, axis=0, keepdims=True)
        mscr[...] = tmax
        sscr[...] = jnp.sum(jnp.exp(vals - tmax), axis=0, keepdims=True)

    def _seg_acc(vals):
        tmax = jnp.max(vals, axis=0, keepdims=True)
        m_old = mscr[...]
        m_new = jnp.maximum(m_old, tmax)
        sscr[...] = (sscr[...] * jnp.exp(m_old - m_new)
                     + jnp.sum(jnp.exp(vals - m_new), axis=0, keepdims=True))
        mscr[...] = m_new

    @pl.when(jnp.logical_not(wr))
    def _compute():
        t = _ctile(i)
        logits = jax.lax.dot_general(
            w_ref[...], x_ref[...], (((1,), (1,)), ((), ())),
            preferred_element_type=jnp.float32)
        logits = logits + jnp.swapaxes(b_ref[...], 0, 1)
        lscr[t % _RING] = logits.astype(jnp.bfloat16)
        row = jax.lax.broadcasted_iota(jnp.int32, (_TW, _N), 0)

        @pl.when(t == 0)
        def _head_init():
            cl = jax.lax.dot_general(
                cw_ref[...], x_ref[...], (((1,), (1,)), ((), ())),
                preferred_element_type=jnp.float32)
            cl = cl + jnp.swapaxes(cb_ref[...], 0, 1)
            cl_scr[...] = cl
            tmax = jnp.max(logits, axis=0, keepdims=True)
            m0 = jnp.maximum(tmax, jnp.max(cl, axis=0, keepdims=True))
            mscr[...] = m0
            sscr[...] = (jnp.sum(jnp.exp(logits - m0), axis=0, keepdims=True)
                         + jnp.sum(jnp.exp(cl - m0), axis=0, keepdims=True))

        @pl.when((t > 0) & (t != _TH) & (t != _TT1) & (t != _NT - 1))
        def _interior():
            _seg_acc(logits)

        @pl.when(t == _TH)
        def _head_end():
            _seg_acc(jnp.where(row < _EH, logits, _NEG))
            zh_scr[...] = mscr[...] + jnp.log(sscr[...])
            _seg_init(jnp.where(row >= _EH, logits, _NEG))

        @pl.when(t == _TT1)
        def _t1_end():
            _seg_acc(jnp.where(row < _ET1, logits, _NEG))
            z1_scr[...] = mscr[...] + jnp.log(sscr[...])
            _seg_init(jnp.where(row >= _ET1, logits, _NEG))

        @pl.when(t == _NT - 1)
        def _t2_end():
            _seg_acc(jnp.where(row < _ET2, logits, _NEG))
            z2_scr[...] = mscr[...] + jnp.log(sscr[...])

    @pl.when(wr)
    def _write():
        wt = _wtile(i)
        cl = cl_scr[...]
        zh = zh_scr[...]
        sub_h = zh
        sub_1 = z1_scr[...] + zh - cl[0:1, :]
        sub_2 = z2_scr[...] + zh - cl[1:2, :]
        rowg = (jax.lax.broadcasted_iota(jnp.int32, (_TW, _N), 0)
                + wt * _TW)
        sub = jnp.where(rowg < _C0, sub_h,
                        jnp.where(rowg < _C1, sub_1, sub_2))
        o_ref[...] = lscr[wt % _RING].astype(jnp.float32) - sub


def kernel(inputs, out_weight, out_bias, cluster_weight, cluster_bias):
    x = inputs.reshape(-1, inputs.shape[-1])
    b2 = out_bias.reshape(1, _C2)
    cb2 = cluster_bias.reshape(1, 2)
    out_t = pl.pallas_call(
        _body,
        grid=(2 * _NT,),
        in_specs=[
            pl.BlockSpec((_N, _D), lambda i: (0, 0)),
            pl.BlockSpec((_TW, _D), _w_map),
            pl.BlockSpec((1, _TW), _b_map),
            pl.BlockSpec((2, _D), lambda i: (0, 0)),
            pl.BlockSpec((1, 2), lambda i: (0, 0)),
        ],
        out_specs=pl.BlockSpec((_TW, _N), _o_map),
        out_shape=jax.ShapeDtypeStruct((_C2, _N), jnp.float32),
        scratch_shapes=[
            pltpu.VMEM((_RING, _TW, _N), jnp.bfloat16),
            pltpu.VMEM((1, _N), jnp.float32),
            pltpu.VMEM((1, _N), jnp.float32),
            pltpu.VMEM((1, _N), jnp.float32),
            pltpu.VMEM((1, _N), jnp.float32),
            pltpu.VMEM((1, _N), jnp.float32),
            pltpu.VMEM((2, _N), jnp.float32),
        ],
        compiler_params=pltpu.CompilerParams(
            dimension_semantics=("arbitrary",)),
    )(x, out_weight, b2, cluster_weight, cb2)
    return out_t.T


# speedup vs baseline: 1.0877x; 1.0877x over previous
import jax
import jax.numpy as jnp
from jax.experimental import pallas as pl
from jax.experimental.pallas import tpu as pltpu

_D = 1024
_N = 128
_C0, _C1, _C2 = 20000, 50000, 100000
_TW = 3072
_NT = 33
_RING = 17
_NEG = -1e30
_TH = 6
_TT1 = 16
_EH = _C0 - _TH * _TW
_ET1 = _C1 - _TT1 * _TW
_ET2 = _C2 - (_NT - 1) * _TW
_STEPS = 50


def _is_write(i):
    return ((i >= 7) & (i < 13)) | ((i >= 17) & (i < 27)) | (i >= 33)


def _ctile(i):
    return jnp.clip(i, 0, _NT - 1)


def _wtile(i):
    t = jnp.where(i < 13, i - 7,
        jnp.where(i < 27, i - 11, i - 17))
    return jnp.clip(t, 0, _NT - 1)


def _w_map(i):
    return (_ctile(i), 0)


def _b_map(i):
    return (0, _ctile(i))


def _o_map(i):
    o = jnp.where(i < 7, 0,
        jnp.where(i < 13, i - 7,
        jnp.where(i < 17, _TH,
        jnp.where(i < 27, i - 11,
        jnp.where(i < 33, _TT1, i - 17)))))
    return (o, 0)


def _body(x_ref, w_ref, b_ref, cw_ref, cb_ref, o_ref,
          lscr, mscr, sscr, zh_scr, z1_scr, z2_scr, cl_scr):
    i = pl.program_id(0)

    def _seg_init(vals):
        tmax = jnp.max(vals, axis=0, keepdims=True)
        mscr[...] = tmax
        sscr[...] = jnp.sum(jnp.exp(vals - tmax), axis=0, keepdims=True)

    def _seg_acc(vals):
        tmax = jnp.max(vals, axis=0, keepdims=True)
        m_old = mscr[...]
        m_new = jnp.maximum(m_old, tmax)
        sscr[...] = (sscr[...] * jnp.exp(m_old - m_new)
                     + jnp.sum(jnp.exp(vals - m_new), axis=0, keepdims=True))
        mscr[...] = m_new

    @pl.when(i < _NT)
    def _compute():
        t = i
        logits = jax.lax.dot_general(
            w_ref[...], x_ref[...], (((1,), (1,)), ((), ())),
            preferred_element_type=jnp.float32)
        logits = logits + jnp.swapaxes(b_ref[...], 0, 1)
        lscr[t % _RING] = logits
        row = jax.lax.broadcasted_iota(jnp.int32, (_TW, _N), 0)

        @pl.when(t == 0)
        def _head_init():
            cl = jax.lax.dot_general(
                cw_ref[...], x_ref[...], (((1,), (1,)), ((), ())),
                preferred_element_type=jnp.float32)
            cl = cl + jnp.swapaxes(cb_ref[...], 0, 1)
            cl_scr[...] = cl
            tmax = jnp.max(logits, axis=0, keepdims=True)
            m0 = jnp.maximum(tmax, jnp.max(cl, axis=0, keepdims=True))
            mscr[...] = m0
            sscr[...] = (jnp.sum(jnp.exp(logits - m0), axis=0, keepdims=True)
                         + jnp.sum(jnp.exp(cl - m0), axis=0, keepdims=True))

        @pl.when((t > 0) & (t != _TH) & (t != _TT1) & (t != _NT - 1))
        def _interior():
            _seg_acc(logits)

        @pl.when(t == _TH)
        def _head_end():
            _seg_acc(jnp.where(row < _EH, logits, _NEG))
            zh_scr[...] = mscr[...] + jnp.log(sscr[...])
            _seg_init(jnp.where(row >= _EH, logits, _NEG))

        @pl.when(t == _TT1)
        def _t1_end():
            _seg_acc(jnp.where(row < _ET1, logits, _NEG))
            z1_scr[...] = mscr[...] + jnp.log(sscr[...])
            _seg_init(jnp.where(row >= _ET1, logits, _NEG))

        @pl.when(t == _NT - 1)
        def _t2_end():
            _seg_acc(jnp.where(row < _ET2, logits, _NEG))
            z2_scr[...] = mscr[...] + jnp.log(sscr[...])

    @pl.when(_is_write(i))
    def _write():
        wt = _wtile(i)
        cl = cl_scr[...]
        zh = zh_scr[...]
        sub_h = zh
        sub_1 = z1_scr[...] + zh - cl[0:1, :]
        sub_2 = z2_scr[...] + zh - cl[1:2, :]
        rowg = (jax.lax.broadcasted_iota(jnp.int32, (_TW, _N), 0)
                + wt * _TW)
        sub = jnp.where(rowg < _C0, sub_h,
                        jnp.where(rowg < _C1, sub_1, sub_2))
        o_ref[...] = lscr[wt % _RING] - sub


def kernel(inputs, out_weight, out_bias, cluster_weight, cluster_bias):
    x = inputs.reshape(-1, inputs.shape[-1])
    b2 = out_bias.reshape(1, _C2)
    cb2 = cluster_bias.reshape(1, 2)
    out_t = pl.pallas_call(
        _body,
        grid=(_STEPS,),
        in_specs=[
            pl.BlockSpec((_N, _D), lambda i: (0, 0)),
            pl.BlockSpec((_TW, _D), _w_map),
            pl.BlockSpec((1, _TW), _b_map),
            pl.BlockSpec((2, _D), lambda i: (0, 0)),
            pl.BlockSpec((1, 2), lambda i: (0, 0)),
        ],
        out_specs=pl.BlockSpec((_TW, _N), _o_map),
        out_shape=jax.ShapeDtypeStruct((_C2, _N), jnp.float32),
        scratch_shapes=[
            pltpu.VMEM((_RING, _TW, _N), jnp.float32),
            pltpu.VMEM((1, _N), jnp.float32),
            pltpu.VMEM((1, _N), jnp.float32),
            pltpu.VMEM((1, _N), jnp.float32),
            pltpu.VMEM((1, _N), jnp.float32),
            pltpu.VMEM((1, _N), jnp.float32),
            pltpu.VMEM((2, _N), jnp.float32),
        ],
        compiler_params=pltpu.CompilerParams(
            dimension_semantics=("arbitrary",)),
    )(x, out_weight, b2, cluster_weight, cb2)
    return out_t.T
